# baseline (speedup 1.0000x reference)
import functools

import jax
from jax import lax
from jax.experimental import pallas as pl
from jax.experimental.pallas import tpu as pltpu
from jax.experimental.pallas import tpu_sc as plsc

NC = 2
NS = 16
NW = NC * NS


def _make_sc_kernel(seq_len, b_sz, emb, dtype):
    rows_per_w = seq_len // NW
    chunk_rows = [64, 56, 8]
    chunk_off = [0, 64, 120]
    buf_off = [0, 64, 0]
    assert sum(chunk_rows) == rows_per_w
    mesh = plsc.VectorSubcoreMesh(core_axis_name="c", subcore_axis_name="s")

    @functools.partial(
        pl.kernel,
        out_type=jax.ShapeDtypeStruct((seq_len, b_sz, emb), dtype),
        mesh=mesh,
        scratch_types=[
            pltpu.VMEM((120, emb), dtype),
            pltpu.SemaphoreType.DMA,
            pltpu.SemaphoreType.DMA,
            pltpu.SemaphoreType.DMA,
        ],
    )
    def sc_kernel(w_hbm, out_hbm, buf, rsem, wsem, w0sem):
        wid = lax.axis_index("s") * NC + lax.axis_index("c")
        base = wid * rows_per_w

        def read(c):
            return pltpu.async_copy(
                w_hbm.at[pl.ds(base + chunk_off[c], chunk_rows[c])],
                buf.at[pl.ds(buf_off[c], chunk_rows[c])],
                rsem,
            )

        def writes(c, sem):
            return [
                pltpu.async_copy(
                    buf.at[pl.ds(buf_off[c], chunk_rows[c])],
                    out_hbm.at[pl.ds(base + chunk_off[c], chunk_rows[c]), b],
                    sem,
                )
                for b in range(b_sz)
            ]

        rds = {0: read(0), 1: read(1)}
        rds[0].wait()
        w0 = writes(0, w0sem)
        rds[1].wait()
        w1 = writes(1, wsem)
        for d in w0:
            d.wait()
        rds[2] = read(2)
        rds[2].wait()
        w2 = writes(2, wsem)
        for d in w1 + w2:
            d.wait()

    return sc_kernel


def kernel(inputs, W):
    seq_len, b_sz = inputs.shape
    emb = W.shape[1]
    return _make_sc_kernel(seq_len, b_sz, emb, W.dtype)(W[:seq_len])
